# baseline (speedup 1.0000x reference)
import functools
import math

import jax
import jax.numpy as jnp
from jax import lax
from jax.experimental import pallas as pl
from jax.experimental.pallas import tpu as pltpu
from jax.experimental.pallas import tpu_sc as plsc

HID = 2048
HEADS = 16
HEAD_DIM = 128
DFF = 5632
GAMMA = 0.5
EPS = 1e-6
THETA = 1000000.0

NEG_INF = -1e9


def _router_body(x_ref, w_ref, b_ref, out_ref):
    r = jnp.dot(x_ref[...], w_ref[...],
                preferred_element_type=jnp.float32)[:, 0] + b_ref[0, 0]
    out_ref[...] = r.reshape(8, 128)


def _select_body(k_sel, n_cap, rw_ref, posf_ref, out_ref):
    j = pl.program_id(0)
    rw2 = rw_ref[...]
    bits = lax.bitcast_convert_type(rw2, jnp.uint32)
    neg = (bits >> jnp.uint32(31)) == jnp.uint32(1)
    key = jnp.where(neg, bits ^ jnp.uint32(0xFFFFFFFF),
                    bits | jnp.uint32(0x80000000))
    key3 = key.reshape(2, 16, 128)
    t = jnp.zeros((2, 1, 1), jnp.uint32)
    for b in range(31, -1, -1):
        cand = t | jnp.uint32(1 << b)
        cnt = jnp.sum((key3 >= cand).astype(jnp.int32), axis=(1, 2),
                      keepdims=True)
        t = jnp.where(cnt >= k_sel, cand, t)
    maskb = (key3 >= t).reshape(32, 128)
    mask2 = maskb.astype(jnp.float32)
    ci = lax.broadcasted_iota(jnp.int32, (128, 128), 0)
    cj = lax.broadcasted_iota(jnp.int32, (128, 128), 1)
    lower = (ci <= cj).astype(jnp.float32)
    inrow = jnp.dot(mask2, lower, preferred_element_type=jnp.float32,
                    precision=lax.Precision.HIGHEST)
    rowtot = jnp.sum(mask2, axis=1, keepdims=True)
    ri = lax.broadcasted_iota(jnp.int32, (32, 32), 0)
    rj = lax.broadcasted_iota(jnp.int32, (32, 32), 1)
    strict = (rj < ri).astype(jnp.float32)
    rowpref = jnp.dot(strict, rowtot, preferred_element_type=jnp.float32,
                      precision=lax.Precision.HIGHEST)
    csum = (inrow + rowpref).astype(jnp.int32)
    written = maskb & (csum <= n_cap)
    cw = jnp.minimum(csum, n_cap)
    sidx_i = (lax.broadcasted_iota(jnp.int32, (32, 128), 0) * 128
              + lax.broadcasted_iota(jnp.int32, (32, 128), 1))
    ranku = (sidx_i + 1) - cw
    jv = ((j * 512 + 1)
          + lax.broadcasted_iota(jnp.int32, (1, 1, 512), 2))
    writtenf = written.astype(jnp.float32)
    onehot = ((csum[:, :, None] == jv) & (writtenf[:, :, None] > 0.5)
              ).astype(jnp.float32)
    onehot_u = ((ranku[:, :, None] == jv) & (writtenf[:, :, None] < 0.5)
                ).astype(jnp.float32)
    sidx = sidx_i.astype(jnp.float32)
    zero = jnp.zeros((32, 128), jnp.float32)
    vals = jnp.stack([sidx, rw2, posf_ref[...], zero, zero, zero, zero,
                      zero], axis=0)
    vals_u = jnp.stack([zero, zero, zero, sidx, zero, zero, zero,
                        zero], axis=0)
    acc = jnp.zeros((8, 512), jnp.float32)
    for r in range(32):
        acc = acc + jnp.dot(vals[:, r, :], onehot[r],
                            preferred_element_type=jnp.float32,
                            precision=lax.Precision.HIGHEST)
        acc = acc + jnp.dot(vals_u[:, r, :], onehot_u[r],
                            preferred_element_type=jnp.float32,
                            precision=lax.Precision.HIGHEST)
    out_ref[...] = acc


def _make_sc_gather(n_rows, n_sel, d):
    mesh = plsc.VectorSubcoreMesh(core_axis_name="c", subcore_axis_name="s")
    nw = 32
    per_w = n_sel // nw
    chunk = 32

    @functools.partial(
        pl.kernel, mesh=mesh,
        out_type=jax.ShapeDtypeStruct((n_sel, d), jnp.float32),
        scratch_types=[
            pltpu.VMEM((chunk,), jnp.int32),
            pltpu.VMEM((chunk, d), jnp.float32),
            pltpu.SemaphoreType.DMA,
        ],
    )
    def sc_gather(hid_hbm, ids_hbm, out_hbm, idx_v, rows_v, sem):
        wid = lax.axis_index("s") * 2 + lax.axis_index("c")
        base = wid * per_w
        for c in range(per_w // chunk):
            off = base + c * chunk
            pltpu.sync_copy(ids_hbm.at[pl.ds(off, chunk)], idx_v)
            pltpu.async_copy(hid_hbm.at[idx_v], rows_v, sem).wait()
            pltpu.sync_copy(rows_v, out_hbm.at[pl.ds(off, chunk)])

    return sc_gather


def _make_sc_scatter(n_rows, n_sel, d):
    mesh = plsc.VectorSubcoreMesh(core_axis_name="c", subcore_axis_name="s")
    nw = 32
    per_w = n_sel // nw
    chunk = 32

    @functools.partial(
        pl.kernel, mesh=mesh,
        out_type=jax.ShapeDtypeStruct((n_rows, d), jnp.float32),
        scratch_types=[
            pltpu.VMEM((chunk,), jnp.int32),
            pltpu.VMEM((chunk, d), jnp.float32),
            pltpu.SemaphoreType.DMA,
        ],
    )
    def sc_scatter(hid_hbm, upd_hbm, sel_ids_hbm, uns_ids_hbm, out_hbm,
                   idx_v, rows_v, sem):
        wid = lax.axis_index("s") * 2 + lax.axis_index("c")
        base = wid * per_w
        for c in range(per_w // chunk):
            off = base + c * chunk
            pltpu.sync_copy(uns_ids_hbm.at[pl.ds(off, chunk)], idx_v)
            pltpu.async_copy(hid_hbm.at[idx_v], rows_v, sem).wait()
            pltpu.async_copy(rows_v, out_hbm.at[idx_v], sem).wait()
        for c in range(per_w // chunk):
            off = base + c * chunk
            pltpu.sync_copy(sel_ids_hbm.at[pl.ds(off, chunk)], idx_v)
            pltpu.sync_copy(upd_hbm.at[pl.ds(off, chunk)], rows_v)
            pltpu.async_copy(rows_v, out_hbm.at[idx_v], sem).wait()

    return sc_scatter


def _qkv_body(x_ref, pos_ref, wq_ref, wk_ref, wv_ref, bq_ref, bk_ref,
              bv_ref, ln1_ref, q_ref, k_ref, v_ref):
    x = x_ref[...]
    h = x * lax.rsqrt(jnp.mean(x * x, axis=1, keepdims=True) + EPS)
    h = (h * ln1_ref[...]).astype(jnp.bfloat16)
    q = jnp.dot(h, wq_ref[...], preferred_element_type=jnp.float32)
    k = jnp.dot(h, wk_ref[...], preferred_element_type=jnp.float32)
    v = jnp.dot(h, wv_ref[...], preferred_element_type=jnp.float32)
    q = q + bq_ref[...]
    k = k + bk_ref[...]
    v = v + bv_ref[...]
    pos = pos_ref[...]
    io = lax.broadcasted_iota(jnp.int32, (1, HEAD_DIM // 2), 1
                              ).astype(jnp.float32)
    inv = jnp.exp(io * jnp.float32(-2.0 * math.log(THETA) / HEAD_DIM))
    ang = pos * inv
    cos = jnp.cos(ang)
    sin = jnp.sin(ang)
    cs = jnp.concatenate([cos, cos], axis=1)[:, None, :]
    sn = jnp.concatenate([sin, sin], axis=1)[:, None, :]

    def rope(m):
        m3 = m.reshape(512, 4, HEAD_DIM)
        rot = jnp.concatenate([-m3[..., HEAD_DIM // 2:],
                               m3[..., :HEAD_DIM // 2]], axis=-1)
        return (m3 * cs + rot * sn).reshape(512, 512)

    q_ref[...] = rope(q).astype(jnp.bfloat16)
    k_ref[...] = rope(k).astype(jnp.bfloat16)
    v_ref[...] = v.astype(jnp.bfloat16)


def _attn_body(qb, q_ref, k_ref, v_ref, o_ref):
    q = q_ref[...]
    k = k_ref[...]
    s = lax.dot_general(q, k, (((1,), (1,)), ((), ())),
                        preferred_element_type=jnp.float32)
    s = s * jnp.float32(1.0 / math.sqrt(HEAD_DIM))
    kl = k.shape[0]
    row = qb * 512 + lax.broadcasted_iota(jnp.int32, (512, kl), 0)
    col = lax.broadcasted_iota(jnp.int32, (512, kl), 1)
    s = jnp.where(col <= row, s, jnp.float32(NEG_INF))
    m = jnp.max(s, axis=1, keepdims=True)
    p = jnp.exp(s - m)
    p = (p / jnp.sum(p, axis=1, keepdims=True)).astype(jnp.bfloat16)
    o_ref[...] = jnp.dot(p, v_ref[...], preferred_element_type=jnp.float32
                         ).astype(jnp.bfloat16)


def _oproj_body(a_ref, wo_ref, x_ref, ln2_ref, x2_ref, h2_ref):
    x2 = x_ref[...] + jnp.dot(a_ref[...], wo_ref[...],
                              preferred_element_type=jnp.float32)
    x2_ref[...] = x2
    h2 = x2 * lax.rsqrt(jnp.mean(x2 * x2, axis=1, keepdims=True) + EPS)
    h2_ref[...] = (h2 * ln2_ref[...]).astype(jnp.bfloat16)


def _gate_body(h2_ref, wg_ref, wu_ref, a_ref):
    h2 = h2_ref[...]
    g = jnp.dot(h2, wg_ref[...], preferred_element_type=jnp.float32)
    u = jnp.dot(h2, wu_ref[...], preferred_element_type=jnp.float32)
    a_ref[...] = (g * u / (1.0 + jnp.exp(-g))).astype(jnp.bfloat16)


def _down_body(a_ref, wd_ref, o_ref):
    kt = pl.program_id(0)
    part = jnp.dot(a_ref[...], wd_ref[...], preferred_element_type=jnp.float32)

    @pl.when(kt == 0)
    def _():
        o_ref[...] = part

    @pl.when(kt > 0)
    def _():
        o_ref[...] = o_ref[...] + part


def _final_body(sel_ref, x2_ref, mlp_ref, rw_ref, o_ref):
    sel = sel_ref[...]
    delta = (x2_ref[...] - sel) + mlp_ref[...]
    o_ref[...] = sel + delta * rw_ref[...]


def kernel(hidden_states, position_ids, W_router, b_router, Wq, bq, Wk, bk,
           Wv, bv, Wo, ln1, ln2, Wg, Wu, Wd):
    B, S, D = hidden_states.shape
    k_sel = max(1, int(GAMMA * S))
    n_tok = B * S
    n_sel = B * k_sel
    hid_flat = hidden_states.reshape(n_tok, D)

    rw2 = pl.pallas_call(
        _router_body,
        grid=(4,),
        in_specs=[
            pl.BlockSpec((n_tok // 4, D), lambda i: (i, 0)),
            pl.BlockSpec((D, 1), lambda i: (0, 0)),
            pl.BlockSpec((1, 1), lambda i: (0, 0)),
        ],
        out_specs=pl.BlockSpec((8, 128), lambda i: (i, 0)),
        out_shape=jax.ShapeDtypeStruct((n_tok // 128, 128), jnp.float32),
    )(hid_flat, W_router, b_router.reshape(1, 1))

    posf = position_ids.astype(jnp.float32).reshape(n_tok // 128, 128)
    sel3 = pl.pallas_call(
        functools.partial(_select_body, k_sel, n_sel),
        grid=(n_sel // 512,),
        in_specs=[
            pl.BlockSpec((n_tok // 128, 128), lambda j: (0, 0)),
            pl.BlockSpec((n_tok // 128, 128), lambda j: (0, 0)),
        ],
        out_specs=pl.BlockSpec((8, 512), lambda j: (0, j)),
        out_shape=jax.ShapeDtypeStruct((8, n_sel), jnp.float32),
    )(rw2, posf)

    flat_ids = sel3[0].astype(jnp.int32)
    sel_rw = sel3[1].reshape(n_sel, 1)
    sel_posf = sel3[2].reshape(n_sel, 1)
    uns_ids = sel3[3].astype(jnp.int32)

    sel = _make_sc_gather(n_tok, n_sel, D)(hid_flat, flat_ids)

    N = n_sel
    nh = HEADS * HEAD_DIM
    wspec = lambda nt, rb: (0, nt)
    bspec = lambda nt, rb: (0, nt)
    q, k, v = pl.pallas_call(
        _qkv_body,
        grid=(nh // 512, N // 512),
        in_specs=[
            pl.BlockSpec((512, D), lambda nt, rb: (rb, 0)),
            pl.BlockSpec((512, 1), lambda nt, rb: (rb, 0)),
            pl.BlockSpec((D, 512), wspec),
            pl.BlockSpec((D, 512), wspec),
            pl.BlockSpec((D, 512), wspec),
            pl.BlockSpec((1, 512), bspec),
            pl.BlockSpec((1, 512), bspec),
            pl.BlockSpec((1, 512), bspec),
            pl.BlockSpec((1, D), lambda nt, rb: (0, 0)),
        ],
        out_specs=[
            pl.BlockSpec((512, 512), lambda nt, rb: (rb, nt)),
            pl.BlockSpec((512, 512), lambda nt, rb: (rb, nt)),
            pl.BlockSpec((512, 512), lambda nt, rb: (rb, nt)),
        ],
        out_shape=[jax.ShapeDtypeStruct((N, nh), jnp.bfloat16)] * 3,
    )(sel, sel_posf, Wq.astype(jnp.bfloat16), Wk.astype(jnp.bfloat16),
      Wv.astype(jnp.bfloat16), bq.reshape(1, nh), bk.reshape(1, nh),
      bv.reshape(1, nh), ln1.reshape(1, D))

    attn_parts = []
    for qb in range(N // 512):
        kl = (qb + 1) * 512
        part = pl.pallas_call(
            functools.partial(_attn_body, qb),
            grid=(HEADS,),
            in_specs=[
                pl.BlockSpec((512, HEAD_DIM), lambda h, _qb=qb: (_qb, h)),
                pl.BlockSpec((kl, HEAD_DIM), lambda h: (0, h)),
                pl.BlockSpec((kl, HEAD_DIM), lambda h: (0, h)),
            ],
            out_specs=pl.BlockSpec((512, HEAD_DIM), lambda h: (0, h)),
            out_shape=jax.ShapeDtypeStruct((512, nh), jnp.bfloat16),
        )(q, k, v)
        attn_parts.append(part)
    attn = jnp.concatenate(attn_parts, axis=0)

    x2, h2 = pl.pallas_call(
        _oproj_body,
        grid=(N // 512,),
        in_specs=[
            pl.BlockSpec((512, nh), lambda i: (i, 0)),
            pl.BlockSpec((nh, D), lambda i: (0, 0)),
            pl.BlockSpec((512, D), lambda i: (i, 0)),
            pl.BlockSpec((1, D), lambda i: (0, 0)),
        ],
        out_specs=[
            pl.BlockSpec((512, D), lambda i: (i, 0)),
            pl.BlockSpec((512, D), lambda i: (i, 0)),
        ],
        out_shape=[jax.ShapeDtypeStruct((N, D), jnp.float32),
                   jax.ShapeDtypeStruct((N, D), jnp.bfloat16)],
    )(attn, Wo.astype(jnp.bfloat16), sel, ln2.reshape(1, D))

    act = pl.pallas_call(
        _gate_body,
        grid=(DFF // 512,),
        in_specs=[
            pl.BlockSpec((N, D), lambda nt: (0, 0)),
            pl.BlockSpec((D, 512), lambda nt: (0, nt)),
            pl.BlockSpec((D, 512), lambda nt: (0, nt)),
        ],
        out_specs=pl.BlockSpec((N, 512), lambda nt: (0, nt)),
        out_shape=jax.ShapeDtypeStruct((N, DFF), jnp.bfloat16),
    )(h2, Wg.astype(jnp.bfloat16), Wu.astype(jnp.bfloat16))

    mlp = pl.pallas_call(
        _down_body,
        grid=(DFF // 512,),
        in_specs=[
            pl.BlockSpec((N, 512), lambda kt: (0, kt)),
            pl.BlockSpec((512, D), lambda kt: (kt, 0)),
        ],
        out_specs=pl.BlockSpec((N, D), lambda kt: (0, 0)),
        out_shape=jax.ShapeDtypeStruct((N, D), jnp.float32),
    )(act, Wd.astype(jnp.bfloat16))

    updated = pl.pallas_call(
        _final_body,
        grid=(N // 512,),
        in_specs=[
            pl.BlockSpec((512, D), lambda i: (i, 0)),
            pl.BlockSpec((512, D), lambda i: (i, 0)),
            pl.BlockSpec((512, D), lambda i: (i, 0)),
            pl.BlockSpec((512, 1), lambda i: (i, 0)),
        ],
        out_specs=pl.BlockSpec((512, D), lambda i: (i, 0)),
        out_shape=jax.ShapeDtypeStruct((N, D), jnp.float32),
    )(sel, x2, mlp, sel_rw)

    out = _make_sc_scatter(n_tok, n_sel, D)(hid_flat, updated, flat_ids,
                                            uns_ids)
    return out.reshape(B, S, D)


# speedup vs baseline: 1.1046x; 1.1046x over previous
import functools
import math

import jax
import jax.numpy as jnp
from jax import lax
from jax.experimental import pallas as pl
from jax.experimental.pallas import tpu as pltpu
from jax.experimental.pallas import tpu_sc as plsc

HID = 2048
HEADS = 16
HEAD_DIM = 128
DFF = 5632
GAMMA = 0.5
EPS = 1e-6
THETA = 1000000.0

NEG_INF = -1e9


def _router_body(x_ref, w_ref, b_ref, out_ref):
    r = jnp.dot(x_ref[...], w_ref[...],
                preferred_element_type=jnp.float32)[:, 0] + b_ref[0, 0]
    out_ref[...] = r.reshape(8, 128)


def _select_body(k_sel, n_cap, rw_ref, posf_ref, out_ref):
    j = pl.program_id(0)
    rw2 = rw_ref[...]
    bits = lax.bitcast_convert_type(rw2, jnp.uint32)
    neg = (bits >> jnp.uint32(31)) == jnp.uint32(1)
    key = jnp.where(neg, bits ^ jnp.uint32(0xFFFFFFFF),
                    bits | jnp.uint32(0x80000000))
    key3 = key.reshape(2, 16, 128)
    t = jnp.zeros((2, 1, 1), jnp.uint32)
    for b in range(31, -1, -1):
        cand = t | jnp.uint32(1 << b)
        cnt = jnp.sum((key3 >= cand).astype(jnp.int32), axis=(1, 2),
                      keepdims=True)
        t = jnp.where(cnt >= k_sel, cand, t)
    maskb = (key3 >= t).reshape(32, 128)
    mask2 = maskb.astype(jnp.float32)
    ci = lax.broadcasted_iota(jnp.int32, (128, 128), 0)
    cj = lax.broadcasted_iota(jnp.int32, (128, 128), 1)
    lower = (ci <= cj).astype(jnp.float32)
    inrow = jnp.dot(mask2, lower, preferred_element_type=jnp.float32,
                    precision=lax.Precision.HIGHEST)
    rowtot = jnp.sum(mask2, axis=1, keepdims=True)
    ri = lax.broadcasted_iota(jnp.int32, (32, 32), 0)
    rj = lax.broadcasted_iota(jnp.int32, (32, 32), 1)
    strict = (rj < ri).astype(jnp.float32)
    rowpref = jnp.dot(strict, rowtot, preferred_element_type=jnp.float32,
                      precision=lax.Precision.HIGHEST)
    csum = (inrow + rowpref).astype(jnp.int32)
    written = maskb & (csum <= n_cap)
    cw = jnp.minimum(csum, n_cap)
    sidx_i = (lax.broadcasted_iota(jnp.int32, (32, 128), 0) * 128
              + lax.broadcasted_iota(jnp.int32, (32, 128), 1))
    ranku = (sidx_i + 1) - cw
    jv = ((j * 512 + 1)
          + lax.broadcasted_iota(jnp.int32, (1, 1, 512), 2))
    writtenf = written.astype(jnp.float32)
    onehot = ((csum[:, :, None] == jv) & (writtenf[:, :, None] > 0.5)
              ).astype(jnp.float32)
    onehot_u = ((ranku[:, :, None] == jv) & (writtenf[:, :, None] < 0.5)
                ).astype(jnp.float32)
    sidx = sidx_i.astype(jnp.float32)
    zero = jnp.zeros((32, 128), jnp.float32)
    vals = jnp.stack([sidx, rw2, posf_ref[...], zero, zero, zero, zero,
                      zero], axis=0)
    vals_u = jnp.stack([zero, zero, zero, sidx, zero, zero, zero,
                        zero], axis=0)
    acc = jnp.zeros((8, 512), jnp.float32)
    for r in range(32):
        acc = acc + jnp.dot(vals[:, r, :], onehot[r],
                            preferred_element_type=jnp.float32,
                            precision=lax.Precision.HIGHEST)
        acc = acc + jnp.dot(vals_u[:, r, :], onehot_u[r],
                            preferred_element_type=jnp.float32,
                            precision=lax.Precision.HIGHEST)
    out_ref[...] = acc


def _make_sc_gather(n_rows, n_sel, d):
    mesh = plsc.VectorSubcoreMesh(core_axis_name="c", subcore_axis_name="s")
    nw = 32
    per_w = n_sel // nw
    chunk = 32

    @functools.partial(
        pl.kernel, mesh=mesh,
        out_type=jax.ShapeDtypeStruct((n_sel, d), jnp.float32),
        scratch_types=[
            pltpu.VMEM((chunk,), jnp.int32),
            pltpu.VMEM((chunk, d), jnp.float32),
            pltpu.SemaphoreType.DMA,
        ],
    )
    def sc_gather(hid_hbm, ids_hbm, out_hbm, idx_v, rows_v, sem):
        wid = lax.axis_index("s") * 2 + lax.axis_index("c")
        base = wid * per_w
        for c in range(per_w // chunk):
            off = base + c * chunk
            pltpu.sync_copy(ids_hbm.at[pl.ds(off, chunk)], idx_v)
            pltpu.async_copy(hid_hbm.at[idx_v], rows_v, sem).wait()
            pltpu.sync_copy(rows_v, out_hbm.at[pl.ds(off, chunk)])

    return sc_gather


def _make_sc_scatter(n_rows, n_sel, d):
    mesh = plsc.VectorSubcoreMesh(core_axis_name="c", subcore_axis_name="s")
    nw = 32
    per_w = n_sel // nw
    chunk = 32

    @functools.partial(
        pl.kernel, mesh=mesh,
        out_type=jax.ShapeDtypeStruct((n_rows, d), jnp.float32),
        scratch_types=[
            pltpu.VMEM((chunk,), jnp.int32),
            pltpu.VMEM((chunk, d), jnp.float32),
            pltpu.SemaphoreType.DMA,
        ],
    )
    def sc_scatter(hid_hbm, upd_hbm, sel_ids_hbm, uns_ids_hbm, out_hbm,
                   idx_v, rows_v, sem):
        wid = lax.axis_index("s") * 2 + lax.axis_index("c")
        base = wid * per_w
        for c in range(per_w // chunk):
            off = base + c * chunk
            pltpu.sync_copy(uns_ids_hbm.at[pl.ds(off, chunk)], idx_v)
            pltpu.async_copy(hid_hbm.at[idx_v], rows_v, sem).wait()
            pltpu.async_copy(rows_v, out_hbm.at[idx_v], sem).wait()
        for c in range(per_w // chunk):
            off = base + c * chunk
            pltpu.sync_copy(sel_ids_hbm.at[pl.ds(off, chunk)], idx_v)
            pltpu.sync_copy(upd_hbm.at[pl.ds(off, chunk)], rows_v)
            pltpu.async_copy(rows_v, out_hbm.at[idx_v], sem).wait()

    return sc_scatter


def _qkv_body(x_ref, pos_ref, wq_ref, wk_ref, wv_ref, bq_ref, bk_ref,
              bv_ref, ln1_ref, q_ref, k_ref, v_ref):
    x = x_ref[...]
    h = x * lax.rsqrt(jnp.mean(x * x, axis=1, keepdims=True) + EPS)
    h = h * ln1_ref[...]
    q = jnp.dot(h, wq_ref[...], preferred_element_type=jnp.float32)
    k = jnp.dot(h, wk_ref[...], preferred_element_type=jnp.float32)
    v = jnp.dot(h, wv_ref[...], preferred_element_type=jnp.float32)
    q = q + bq_ref[...]
    k = k + bk_ref[...]
    v = v + bv_ref[...]
    pos = pos_ref[...]
    io = lax.broadcasted_iota(jnp.int32, (1, HEAD_DIM // 2), 1
                              ).astype(jnp.float32)
    inv = jnp.exp(io * jnp.float32(-2.0 * math.log(THETA) / HEAD_DIM))
    ang = pos * inv
    cos = jnp.cos(ang)
    sin = jnp.sin(ang)
    cs = jnp.concatenate([cos, cos], axis=1)[:, None, :]
    sn = jnp.concatenate([sin, sin], axis=1)[:, None, :]

    def rope(m):
        m3 = m.reshape(512, 4, HEAD_DIM)
        rot = jnp.concatenate([-m3[..., HEAD_DIM // 2:],
                               m3[..., :HEAD_DIM // 2]], axis=-1)
        return (m3 * cs + rot * sn).reshape(512, 512)

    q_ref[...] = rope(q)
    k_ref[...] = rope(k)
    v_ref[...] = v


def _attn_body(qb, q_ref, k_ref, v_ref, o_ref):
    q = q_ref[...]
    k = k_ref[...]
    s = lax.dot_general(q, k, (((1,), (1,)), ((), ())),
                        preferred_element_type=jnp.float32)
    s = s * jnp.float32(1.0 / math.sqrt(HEAD_DIM))
    kl = k.shape[0]
    row = qb * 512 + lax.broadcasted_iota(jnp.int32, (512, kl), 0)
    col = lax.broadcasted_iota(jnp.int32, (512, kl), 1)
    s = jnp.where(col <= row, s, jnp.float32(NEG_INF))
    m = jnp.max(s, axis=1, keepdims=True)
    p = jnp.exp(s - m)
    p = p / jnp.sum(p, axis=1, keepdims=True)
    o_ref[...] = jnp.dot(p, v_ref[...], preferred_element_type=jnp.float32)


def _oproj_body(a_ref, wo_ref, x_ref, ln2_ref, x2_ref, h2_ref):
    x2 = x_ref[...] + jnp.dot(a_ref[...], wo_ref[...],
                              preferred_element_type=jnp.float32)
    x2_ref[...] = x2
    h2 = x2 * lax.rsqrt(jnp.mean(x2 * x2, axis=1, keepdims=True) + EPS)
    h2_ref[...] = h2 * ln2_ref[...]


def _gate_body(h2_ref, wg_ref, wu_ref, a_ref):
    h2 = h2_ref[...]
    g = jnp.dot(h2, wg_ref[...], preferred_element_type=jnp.float32)
    u = jnp.dot(h2, wu_ref[...], preferred_element_type=jnp.float32)
    a_ref[...] = g * u / (1.0 + jnp.exp(-g))


def _down_body(a_ref, wd_ref, o_ref):
    kt = pl.program_id(0)
    part = jnp.dot(a_ref[...], wd_ref[...], preferred_element_type=jnp.float32)

    @pl.when(kt == 0)
    def _():
        o_ref[...] = part

    @pl.when(kt > 0)
    def _():
        o_ref[...] = o_ref[...] + part


def _final_body(sel_ref, x2_ref, mlp_ref, rw_ref, o_ref):
    sel = sel_ref[...]
    delta = (x2_ref[...] - sel) + mlp_ref[...]
    o_ref[...] = sel + delta * rw_ref[...]


def kernel(hidden_states, position_ids, W_router, b_router, Wq, bq, Wk, bk,
           Wv, bv, Wo, ln1, ln2, Wg, Wu, Wd):
    B, S, D = hidden_states.shape
    k_sel = max(1, int(GAMMA * S))
    n_tok = B * S
    n_sel = B * k_sel
    hid_flat = hidden_states.reshape(n_tok, D)

    rw2 = pl.pallas_call(
        _router_body,
        grid=(4,),
        in_specs=[
            pl.BlockSpec((n_tok // 4, D), lambda i: (i, 0)),
            pl.BlockSpec((D, 1), lambda i: (0, 0)),
            pl.BlockSpec((1, 1), lambda i: (0, 0)),
        ],
        out_specs=pl.BlockSpec((8, 128), lambda i: (i, 0)),
        out_shape=jax.ShapeDtypeStruct((n_tok // 128, 128), jnp.float32),
    )(hid_flat, W_router, b_router.reshape(1, 1))

    posf = position_ids.astype(jnp.float32).reshape(n_tok // 128, 128)
    sel3 = pl.pallas_call(
        functools.partial(_select_body, k_sel, n_sel),
        grid=(n_sel // 512,),
        in_specs=[
            pl.BlockSpec((n_tok // 128, 128), lambda j: (0, 0)),
            pl.BlockSpec((n_tok // 128, 128), lambda j: (0, 0)),
        ],
        out_specs=pl.BlockSpec((8, 512), lambda j: (0, j)),
        out_shape=jax.ShapeDtypeStruct((8, n_sel), jnp.float32),
    )(rw2, posf)

    flat_ids = sel3[0].astype(jnp.int32)
    sel_rw = sel3[1].reshape(n_sel, 1)
    sel_posf = sel3[2].reshape(n_sel, 1)
    uns_ids = sel3[3].astype(jnp.int32)

    sel = _make_sc_gather(n_tok, n_sel, D)(hid_flat, flat_ids)

    N = n_sel
    nh = HEADS * HEAD_DIM
    wspec = lambda nt, rb: (0, nt)
    bspec = lambda nt, rb: (0, nt)
    q, k, v = pl.pallas_call(
        _qkv_body,
        grid=(nh // 512, N // 512),
        in_specs=[
            pl.BlockSpec((512, D), lambda nt, rb: (rb, 0)),
            pl.BlockSpec((512, 1), lambda nt, rb: (rb, 0)),
            pl.BlockSpec((D, 512), wspec),
            pl.BlockSpec((D, 512), wspec),
            pl.BlockSpec((D, 512), wspec),
            pl.BlockSpec((1, 512), bspec),
            pl.BlockSpec((1, 512), bspec),
            pl.BlockSpec((1, 512), bspec),
            pl.BlockSpec((1, D), lambda nt, rb: (0, 0)),
        ],
        out_specs=[
            pl.BlockSpec((512, 512), lambda nt, rb: (rb, nt)),
            pl.BlockSpec((512, 512), lambda nt, rb: (rb, nt)),
            pl.BlockSpec((512, 512), lambda nt, rb: (rb, nt)),
        ],
        out_shape=[jax.ShapeDtypeStruct((N, nh), jnp.float32)] * 3,
    )(sel, sel_posf, Wq, Wk, Wv, bq.reshape(1, nh), bk.reshape(1, nh),
      bv.reshape(1, nh), ln1.reshape(1, D))

    attn_parts = []
    for qb in range(N // 512):
        kl = (qb + 1) * 512
        part = pl.pallas_call(
            functools.partial(_attn_body, qb),
            grid=(HEADS,),
            in_specs=[
                pl.BlockSpec((512, HEAD_DIM), lambda h, _qb=qb: (_qb, h)),
                pl.BlockSpec((kl, HEAD_DIM), lambda h: (0, h)),
                pl.BlockSpec((kl, HEAD_DIM), lambda h: (0, h)),
            ],
            out_specs=pl.BlockSpec((512, HEAD_DIM), lambda h: (0, h)),
            out_shape=jax.ShapeDtypeStruct((512, nh), jnp.float32),
        )(q, k, v)
        attn_parts.append(part)
    attn = jnp.concatenate(attn_parts, axis=0)

    x2, h2 = pl.pallas_call(
        _oproj_body,
        grid=(N // 512,),
        in_specs=[
            pl.BlockSpec((512, nh), lambda i: (i, 0)),
            pl.BlockSpec((nh, D), lambda i: (0, 0)),
            pl.BlockSpec((512, D), lambda i: (i, 0)),
            pl.BlockSpec((1, D), lambda i: (0, 0)),
        ],
        out_specs=[
            pl.BlockSpec((512, D), lambda i: (i, 0)),
            pl.BlockSpec((512, D), lambda i: (i, 0)),
        ],
        out_shape=[jax.ShapeDtypeStruct((N, D), jnp.float32)] * 2,
    )(attn, Wo, sel, ln2.reshape(1, D))

    act = pl.pallas_call(
        _gate_body,
        grid=(DFF // 512,),
        in_specs=[
            pl.BlockSpec((N, D), lambda nt: (0, 0)),
            pl.BlockSpec((D, 512), lambda nt: (0, nt)),
            pl.BlockSpec((D, 512), lambda nt: (0, nt)),
        ],
        out_specs=pl.BlockSpec((N, 512), lambda nt: (0, nt)),
        out_shape=jax.ShapeDtypeStruct((N, DFF), jnp.float32),
    )(h2, Wg, Wu)

    mlp = pl.pallas_call(
        _down_body,
        grid=(DFF // 512,),
        in_specs=[
            pl.BlockSpec((N, 512), lambda kt: (0, kt)),
            pl.BlockSpec((512, D), lambda kt: (kt, 0)),
        ],
        out_specs=pl.BlockSpec((N, D), lambda kt: (0, 0)),
        out_shape=jax.ShapeDtypeStruct((N, D), jnp.float32),
    )(act, Wd)

    updated = pl.pallas_call(
        _final_body,
        grid=(N // 512,),
        in_specs=[
            pl.BlockSpec((512, D), lambda i: (i, 0)),
            pl.BlockSpec((512, D), lambda i: (i, 0)),
            pl.BlockSpec((512, D), lambda i: (i, 0)),
            pl.BlockSpec((512, 1), lambda i: (i, 0)),
        ],
        out_specs=pl.BlockSpec((512, D), lambda i: (i, 0)),
        out_shape=jax.ShapeDtypeStruct((N, D), jnp.float32),
    )(sel, x2, mlp, sel_rw)

    out = _make_sc_scatter(n_tok, n_sel, D)(hid_flat, updated, flat_ids,
                                            uns_ids)
    return out.reshape(B, S, D)
